# Initial kernel scaffold; baseline (speedup 1.0000x reference)
import jax
import jax.numpy as jnp
from jax.experimental import pallas as pl


def kernel(x, edge_index, edge_weight, W1, b1, W2, b2, g1, be1, g2, be2):
    raise NotImplementedError("write your pallas kernel here")


# speedup vs baseline: 4.6785x; 4.6785x over previous
import functools

import jax
import jax.numpy as jnp
from jax import lax
from jax.experimental import pallas as pl
from jax.experimental.pallas import tpu as pltpu
from jax.experimental.pallas import tpu_sc as plsc

_NC, _NS, _L = 2, 16, 16
_NW = _NC * _NS
_C = 128


@functools.lru_cache(maxsize=None)
def _make_sc_agg(N_pad, D, CH):
    mesh = plsc.VectorSubcoreMesh(core_axis_name="c", subcore_axis_name="s")
    rows_per_tile = N_pad // _NS
    n_full = rows_per_tile // _C

    @functools.partial(
        pl.kernel,
        out_type=jax.ShapeDtypeStruct((_NC, N_pad, D), jnp.float32),
        mesh=mesh,
        scratch_types=[
            pltpu.VMEM((CH, _C), jnp.int32),
            pltpu.VMEM((CH, _C), jnp.int32),
            pltpu.VMEM((CH, _C), jnp.float32),
            pltpu.VMEM((_C, D), jnp.float32),
            pltpu.VMEM_SHARED((N_pad, D), jnp.float32),
            pltpu.SemaphoreType.DMA,
        ],
    )
    def sc_agg(x_hbm, src_hbm, dst_hbm, w_hbm, out_hbm,
               src_v, dst_v, w_v, rows_v, acc_s, sem):
        c = lax.axis_index("c")
        s = lax.axis_index("s")
        wid = c * _NS + s

        pltpu.sync_copy(src_hbm.at[wid], src_v)
        pltpu.sync_copy(dst_hbm.at[wid], dst_v)
        pltpu.sync_copy(w_hbm.at[wid], w_v)

        zeros = jnp.zeros((_L,), jnp.float32)

        def _zrow(i, _):
            for cc in range(D // _L):
                rows_v[i, pl.ds(cc * _L, _L)] = zeros
            return 0

        lax.fori_loop(0, _C, _zrow, 0)
        base = pl.multiple_of(s * rows_per_tile, _C)
        for k in range(n_full):
            pltpu.sync_copy(rows_v, acc_s.at[pl.ds(base + k * _C, _C)])
        plsc.subcore_barrier()

        def _chunk(j, _):
            pltpu.async_copy(x_hbm.at[src_v.at[j]], rows_v, sem).wait()

            def _grp(g, _):
                wv = w_v[j, pl.ds(g * _L, _L)]
                for r in range(_L):
                    ws = wv[r]
                    i = g * _L + r
                    for cc in range(D // _L):
                        sl = pl.ds(cc * _L, _L)
                        rows_v[i, sl] = rows_v[i, sl] * ws
                return 0

            lax.fori_loop(0, _C // _L, _grp, 0)
            pltpu.sync_copy(rows_v, acc_s.at[dst_v.at[j]], add=True)
            return 0

        lax.fori_loop(0, CH, _chunk, 0)

        plsc.subcore_barrier()
        for k in range(n_full):
            sl = pl.ds(base + k * _C, _C)
            pltpu.sync_copy(acc_s.at[sl], out_hbm.at[c, sl])

    return sc_agg


def _mlp_body(p0_ref, p1_ref, W1_ref, b1_ref, W2_ref, b2_ref,
              g1_ref, be1_ref, g2_ref, be2_ref, out_ref):
    agg = p0_ref[...] + p1_ref[...]
    h = jnp.dot(agg, W1_ref[...], preferred_element_type=jnp.float32)
    h = h + b1_ref[...][None, :]
    mu = jnp.mean(h, axis=0, keepdims=True)
    var = jnp.mean((h - mu) ** 2, axis=0, keepdims=True)
    h = g1_ref[...][None, :] * (h - mu) / jnp.sqrt(var + 1e-5) + be1_ref[...][None, :]
    h = jnp.maximum(h, 0.0)
    h = jnp.dot(h, W2_ref[...], preferred_element_type=jnp.float32)
    h = h + b2_ref[...][None, :]
    mu2 = jnp.mean(h, axis=0, keepdims=True)
    var2 = jnp.mean((h - mu2) ** 2, axis=0, keepdims=True)
    h = g2_ref[...][None, :] * (h - mu2) / jnp.sqrt(var2 + 1e-5) + be2_ref[...][None, :]
    out_ref[...] = jnp.maximum(h, 0.0)


def kernel(x, edge_index, edge_weight, W1, b1, W2, b2, g1, be1, g2, be2):
    N, D = x.shape
    E = edge_weight.shape[0]
    CH = -(-E // (_NW * _C))
    pad = _NW * _C * CH - E

    src = edge_index[0]
    dst = edge_index[1]
    w = edge_weight
    if pad:
        src = jnp.concatenate([src, jnp.zeros((pad,), src.dtype)])
        dst = jnp.concatenate([dst, jnp.zeros((pad,), dst.dtype)])
        w = jnp.concatenate([w, jnp.zeros((pad,), w.dtype)])
    src = src.reshape(_NW, CH, _C)
    dst = dst.reshape(_NW, CH, _C)
    w = w.reshape(_NW, CH, _C)

    rpt = -(-N // (_NS * _C)) * _C
    N_pad = rpt * _NS
    partials = _make_sc_agg(N_pad, D, CH)(x, src, dst, w)

    return pl.pallas_call(
        _mlp_body,
        out_shape=jax.ShapeDtypeStruct((N, D), jnp.float32),
    )(partials[0, :N], partials[1, :N], W1, b1, W2, b2, g1, be1, g2, be2)
